# Initial kernel scaffold; baseline (speedup 1.0000x reference)
import functools

import numpy as np
import jax
import jax.numpy as jnp
from jax import lax
from jax.experimental import pallas as pl
from jax.experimental.pallas import tpu as pltpu
from jax.experimental.pallas import tpu_sc as plsc

_B = 8
_NW = 32


def _bilinear_consts(h):
    ys = np.linspace(0.0, h - 1, 2 * h, dtype=np.float32)
    y0 = np.floor(ys).astype(np.int32)
    y1 = np.minimum(y0 + 1, h - 1)
    wy = (ys - y0).astype(np.float32)
    return y0, y1, wy


def _w_upsample_matrix(w):
    x0, x1, wx = _bilinear_consts(w)
    m = np.zeros((w, 2 * w), dtype=np.float32)
    cols = np.arange(2 * w)
    m[x0, cols] += 1.0 - wx
    m[x1, cols] += wx
    return m


def _sc_gather_all(t1, t2, t3, i11, i12, i22, i23):
    mesh = plsc.VectorSubcoreMesh(core_axis_name="c", subcore_axis_name="s")
    out_type = [
        jax.ShapeDtypeStruct((_B * 768, 3136), jnp.float32),
        jax.ShapeDtypeStruct((_B * 1024, 784), jnp.float32),
        jax.ShapeDtypeStruct((_B * 512, 784), jnp.float32),
        jax.ShapeDtypeStruct((_B * 512, 196), jnp.float32),
    ]
    scratch_types = [
        pltpu.VMEM((128,), jnp.int32),
        pltpu.VMEM((8, 3136), jnp.float32),
        pltpu.VMEM((32, 784), jnp.float32),
        pltpu.VMEM((128, 196), jnp.float32),
        pltpu.SemaphoreType.DMA,
    ]

    @functools.partial(pl.kernel, out_type=out_type, mesh=mesh,
                       scratch_types=scratch_types)
    def k(t1h, t2h, t3h, i11h, i12h, i22h, i23h,
          o1h, o2h, g2h, g3h, idxv, buf1, buf2, buf3, sem):
        wid = lax.axis_index("s") * 2 + lax.axis_index("c")
        b = wid // 4
        q = wid % 4

        def task(idx_h, n, src_off, table_h, buf, chunk, out_h, out_base):
            pltpu.sync_copy(idx_h.at[pl.ds(q * n, n)], idxv.at[pl.ds(0, n)])
            for j in range(n // 16):
                sl = pl.ds(j * 16, 16)
                idxv[sl] = idxv[sl] + src_off
            for k0 in range(0, n, chunk):
                pltpu.async_copy(
                    table_h.at[idxv.at[pl.ds(k0, chunk)]], buf, sem).wait()
                pltpu.sync_copy(buf, out_h.at[pl.ds(out_base + k0, chunk)])

        task(i11h, 64, b * 256, t1h, buf1, 8, o1h, b * 768 + q * 64)
        task(i12h, 128, b * 512, t2h, buf2, 32, g2h, wid * 128)
        task(i22h, 128, b * 512, t2h, buf2, 32, o2h, b * 1024 + q * 128)
        task(i23h, 128, b * 1024, t3h, buf3, 128, g3h, wid * 128)

    return k(t1, t2, t3, i11, i12, i22, i23)


def _tc_upsample_into(g, opartial, h, c_skip, cg):
    c_up = g.shape[1]
    y0, y1, wy = _bilinear_consts(h)
    bwt = jnp.asarray(_w_upsample_matrix(h))

    def body(o_alias_ref, g_ref, bw_ref, out_ref):
        del o_alias_ref
        x2 = g_ref[0].reshape(cg * h, h)
        t = jnp.dot(x2, bw_ref[...], preferred_element_type=jnp.float32)
        t3 = t.reshape(cg, h, 2 * h)
        for i in range(2 * h):
            w = float(wy[i])
            if w == 0.0:
                out_ref[0, :, i, :] = t3[:, int(y0[i]), :]
            else:
                out_ref[0, :, i, :] = (t3[:, int(y0[i]), :] * (1.0 - w)
                                       + t3[:, int(y1[i]), :] * w)

    return pl.pallas_call(
        body,
        grid=(_B, c_up // cg),
        in_specs=[
            pl.BlockSpec(memory_space=pltpu.ANY),
            pl.BlockSpec((1, cg, h, h), lambda bi, gi: (bi, gi, 0, 0)),
            pl.BlockSpec((h, 2 * h), lambda bi, gi: (0, 0)),
        ],
        out_specs=pl.BlockSpec((1, cg, 2 * h, 2 * h),
                               lambda bi, gi: (bi, gi + c_skip // cg, 0, 0)),
        out_shape=jax.ShapeDtypeStruct(
            (_B, c_skip + c_up, 2 * h, 2 * h), jnp.float32),
        input_output_aliases={0: 0},
    )(opartial, g, bwt)


def kernel(feat_layer1, feat_layer2, feat_layer3,
           idx_block1_layer1, idx_block1_layer2,
           idx_block2_layer2, idx_block2_layer3):
    t1 = feat_layer1.reshape(_B * 256, 3136)
    t2 = feat_layer2.reshape(_B * 512, 784)
    t3 = feat_layer3.reshape(_B * 1024, 196)
    o1p, o2p, g2, g3 = _sc_gather_all(
        t1, t2, t3, idx_block1_layer1, idx_block1_layer2,
        idx_block2_layer2, idx_block2_layer3)
    block1 = _tc_upsample_into(
        g2.reshape(_B, 512, 28, 28), o1p.reshape(_B, 768, 56, 56),
        h=28, c_skip=256, cg=32)
    block2 = _tc_upsample_into(
        g3.reshape(_B, 512, 14, 14), o2p.reshape(_B, 1024, 28, 28),
        h=14, c_skip=512, cg=32)
    return (block1, block2)


# speedup vs baseline: 3.0578x; 3.0578x over previous
import numpy as np
import jax
import jax.numpy as jnp
from jax.experimental import pallas as pl

_B = 8


def _interp_mat(h):
    ys = np.linspace(0.0, h - 1, 2 * h, dtype=np.float32)
    y0 = np.floor(ys).astype(np.int32)
    y1 = np.minimum(y0 + 1, h - 1)
    wy = (ys - y0).astype(np.float32)
    m = np.zeros((2 * h, h), dtype=np.float32)
    rows = np.arange(2 * h)
    m[rows, y0] += 1.0 - wy
    m[rows, y1] += wy
    return m


def _lane_gather(x2, idx, cin, cout):
    r = x2.shape[0]
    idxb = jnp.broadcast_to((idx & 127)[None, :], (r, cout))
    hi = idx >> 7
    acc = jnp.take_along_axis(x2[:, 0:128], idxb, axis=1)
    for h in range(1, cin // 128):
        g = jnp.take_along_axis(x2[:, h * 128:(h + 1) * 128], idxb, axis=1)
        sel = jnp.broadcast_to((hi == h)[None, :], (r, cout))
        acc = jnp.where(sel, g, acc)
    return acc


def _gather1(p1, i11):
    def body(x_ref, i_ref, o_ref):
        x2 = x_ref[...].reshape(56 * 56, 256)
        g = _lane_gather(x2, i_ref[...], 256, 256)
        o_ref[...] = g.reshape(1, 56, 56, 256)

    return pl.pallas_call(
        body,
        grid=(_B,),
        in_specs=[
            pl.BlockSpec((1, 56, 56, 256), lambda b: (b, 0, 0, 0)),
            pl.BlockSpec((256,), lambda b: (0,)),
        ],
        out_specs=pl.BlockSpec((1, 56, 56, 256), lambda b: (b, 0, 0, 0)),
        out_shape=jax.ShapeDtypeStruct((_B, 56, 56, 768), jnp.float32),
    )(p1, i11)


def _gather2(p2, i12, i22):
    def body(x_ref, ia_ref, ib_ref, g_ref, o_ref):
        x2 = x_ref[...].reshape(28 * 8, 512)
        ga = _lane_gather(x2, ia_ref[...], 512, 512)
        gb = _lane_gather(x2, ib_ref[...], 512, 512)
        g_ref[...] = ga.reshape(1, 28, 8, 512)
        o_ref[...] = gb.reshape(1, 28, 8, 512)

    return pl.pallas_call(
        body,
        grid=(28,),
        in_specs=[
            pl.BlockSpec((1, 28, 8, 512), lambda h: (h, 0, 0, 0)),
            pl.BlockSpec((512,), lambda h: (0,)),
            pl.BlockSpec((512,), lambda h: (0,)),
        ],
        out_specs=[
            pl.BlockSpec((1, 28, 8, 512), lambda h: (h, 0, 0, 0)),
            pl.BlockSpec((1, 28, 8, 512), lambda h: (h, 0, 0, 0)),
        ],
        out_shape=[
            jax.ShapeDtypeStruct((28, 28, _B, 512), jnp.float32),
            jax.ShapeDtypeStruct((28, 28, _B, 1024), jnp.float32),
        ],
    )(p2, i12, i22)


def _gather3(p3, i23):
    def body(x_ref, i_ref, o_ref):
        x2 = x_ref[...].reshape(14 * 8, 1024)
        g = _lane_gather(x2, i_ref[...], 1024, 512)
        o_ref[...] = g.reshape(1, 14, 8, 512)

    return pl.pallas_call(
        body,
        grid=(14,),
        in_specs=[
            pl.BlockSpec((1, 14, 8, 1024), lambda h: (h, 0, 0, 0)),
            pl.BlockSpec((512,), lambda h: (0,)),
        ],
        out_specs=pl.BlockSpec((1, 14, 8, 512), lambda h: (h, 0, 0, 0)),
        out_shape=jax.ShapeDtypeStruct((14, 14, _B, 512), jnp.float32),
    )(p3, i23)


def _upsample1(g2, o1n_partial):
    m2 = jnp.asarray(_interp_mat(28))

    def body(o_alias_ref, top_ref, bot_ref, m_ref, o_ref):
        del o_alias_ref
        i = pl.program_id(1)
        wy = (((i * 27) % 55).astype(jnp.float32) / 55.0).astype(jnp.float32)
        u = top_ref[0] * (1.0 - wy) + bot_ref[0] * wy
        v = jax.lax.dot_general(m_ref[...], u, (((1,), (0,)), ((), ())),
                                preferred_element_type=jnp.float32)
        o_ref[...] = v.transpose(1, 0, 2).reshape(_B, 1, 56, 256)

    return pl.pallas_call(
        body,
        grid=(2, 56),
        in_specs=[
            pl.BlockSpec(memory_space=pl.ANY),
            pl.BlockSpec((1, 28, 8, 256), lambda cq, i: ((i * 27) // 55, 0, 0, cq)),
            pl.BlockSpec((1, 28, 8, 256),
                         lambda cq, i: (jnp.minimum((i * 27) // 55 + 1, 27), 0, 0, cq)),
            pl.BlockSpec((56, 28), lambda cq, i: (0, 0)),
        ],
        out_specs=pl.BlockSpec((_B, 1, 56, 256), lambda cq, i: (0, i, 0, 1 + cq)),
        out_shape=jax.ShapeDtypeStruct((_B, 56, 56, 768), jnp.float32),
        input_output_aliases={0: 0},
    )(o1n_partial, g2, g2, m2)


def _upsample2(g3, o2n_partial):
    m3 = jnp.asarray(_interp_mat(14))

    def body(o_alias_ref, top_ref, bot_ref, m_ref, o_ref):
        del o_alias_ref
        i = pl.program_id(0)
        wy = (((i * 13) % 27).astype(jnp.float32) / 27.0).astype(jnp.float32)
        u = top_ref[0] * (1.0 - wy) + bot_ref[0] * wy
        v = jax.lax.dot_general(m_ref[...], u, (((1,), (0,)), ((), ())),
                                preferred_element_type=jnp.float32)
        o_ref[...] = v.reshape(1, 28, _B, 512)

    return pl.pallas_call(
        body,
        grid=(28,),
        in_specs=[
            pl.BlockSpec(memory_space=pl.ANY),
            pl.BlockSpec((1, 14, 8, 512), lambda i: ((i * 13) // 27, 0, 0, 0)),
            pl.BlockSpec((1, 14, 8, 512),
                         lambda i: (jnp.minimum((i * 13) // 27 + 1, 13), 0, 0, 0)),
            pl.BlockSpec((28, 14), lambda i: (0, 0)),
        ],
        out_specs=pl.BlockSpec((1, 28, _B, 512), lambda i: (i, 0, 0, 1)),
        out_shape=jax.ShapeDtypeStruct((28, 28, _B, 1024), jnp.float32),
        input_output_aliases={0: 0},
    )(o2n_partial, g3, g3, m3)


def kernel(feat_layer1, feat_layer2, feat_layer3,
           idx_block1_layer1, idx_block1_layer2,
           idx_block2_layer2, idx_block2_layer3):
    p1 = feat_layer1.transpose(0, 2, 3, 1)
    p2 = feat_layer2.transpose(2, 3, 0, 1)
    p3 = feat_layer3.transpose(2, 3, 0, 1)

    o1p = _gather1(p1, idx_block1_layer1)
    g2, o2p = _gather2(p2, idx_block1_layer2, idx_block2_layer2)
    g3 = _gather3(p3, idx_block2_layer3)

    o1n = _upsample1(g2, o1p)
    o2n = _upsample2(g3, o2p)

    block1 = o1n.transpose(0, 3, 1, 2)
    block2 = o2n.transpose(2, 3, 0, 1)
    return (block1, block2)
